# baseline (speedup 1.0000x reference)
import functools

import jax
import jax.numpy as jnp
from jax import lax
from jax.experimental import pallas as pl
from jax.experimental.pallas import tpu as pltpu
from jax.experimental.pallas import tpu_sc as plsc

_NC = 2
_NS = 16
_NW = _NC * _NS
_CHUNK = 128


def _sc_gather_body(idx_hbm, table_hbm, out_hbm, idx_v, rows_v, sem):
    wid = lax.axis_index("s") * _NC + lax.axis_index("c")
    n_chunks = idx_v.shape[0]
    rows_per_w = n_chunks * _CHUNK
    pltpu.sync_copy(idx_hbm.at[wid], idx_v)
    copies = [
        pltpu.async_copy(
            table_hbm.at[idx_v.at[c]],
            rows_v.at[pl.ds(c * _CHUNK, _CHUNK)],
            sem,
        )
        for c in range(n_chunks)
    ]
    for cp in copies:
        cp.wait()
    pltpu.sync_copy(rows_v, out_hbm.at[pl.ds(wid * rows_per_w, rows_per_w)])


def _sc_gather(user_id, user_table):
    batch, dim = user_id.shape[0], user_table.shape[1]
    rows_per_w = batch // _NW
    n_chunks = rows_per_w // _CHUNK
    idx3 = user_id.astype(jnp.int32).reshape(_NW, n_chunks, _CHUNK)
    mesh = plsc.VectorSubcoreMesh(core_axis_name="c", subcore_axis_name="s")
    gather = pl.kernel(
        _sc_gather_body,
        out_type=jax.ShapeDtypeStruct((batch, dim), jnp.float32),
        mesh=mesh,
        scratch_types=[
            pltpu.VMEM((n_chunks, _CHUNK), jnp.int32),
            pltpu.VMEM((rows_per_w, dim), jnp.float32),
            pltpu.SemaphoreType.DMA,
        ],
        compiler_params=pltpu.CompilerParams(use_tc_tiling_on_sc=False),
    )
    return gather(idx3, user_table)


def _tc_body(u_ref, tm_ref, ts_ref, wu_ref, mc_ref, wt_ref, o_ref):
    tm = tm_ref[0, 0, :]
    n_modes = mc_ref.shape[0]
    onehot = (
        tm[:, None] == lax.broadcasted_iota(jnp.int32, (1, n_modes), 1)
    ).astype(jnp.float32)
    acc = jnp.dot(u_ref[...], wu_ref[...], preferred_element_type=jnp.float32)
    acc += jnp.dot(onehot, mc_ref[...], preferred_element_type=jnp.float32)
    acc += jnp.dot(ts_ref[...], wt_ref[...], preferred_element_type=jnp.float32)
    o_ref[...] = acc


def kernel(user_id, transport_mode, timestamp, user_table, mode_table,
           W_time, b_time, W_pref, b_pref):
    batch = user_id.shape[0]
    dim = user_table.shape[1]
    out_dim = W_pref.shape[0]
    n_modes = mode_table.shape[0]
    t_in = timestamp.shape[1]

    Wu = W_pref[:, :dim]
    Wm = W_pref[:, dim:2 * dim]
    Wt = W_pref[:, 2 * dim:3 * dim]
    mode_lut = mode_table @ Wm.T + b_pref + b_time @ Wt.T
    t_pad = 8
    W_ts = jnp.zeros((t_pad, out_dim), jnp.float32).at[:t_in].set((Wt @ W_time).T)
    ts_pad = jnp.zeros((batch, t_pad), jnp.float32).at[:, :t_in].set(timestamp)

    user_emb = _sc_gather(user_id, user_table)

    blk = 2048
    n_blk = batch // blk
    tm3 = transport_mode.astype(jnp.int32).reshape(n_blk, 1, blk)

    return pl.pallas_call(
        _tc_body,
        grid=(n_blk,),
        in_specs=[
            pl.BlockSpec((blk, dim), lambda i: (i, 0)),
            pl.BlockSpec((1, 1, blk), lambda i: (i, 0, 0)),
            pl.BlockSpec((blk, t_pad), lambda i: (i, 0)),
            pl.BlockSpec((dim, out_dim), lambda i: (0, 0)),
            pl.BlockSpec((n_modes, out_dim), lambda i: (0, 0)),
            pl.BlockSpec((t_pad, out_dim), lambda i: (0, 0)),
        ],
        out_specs=pl.BlockSpec((blk, out_dim), lambda i: (i, 0)),
        out_shape=jax.ShapeDtypeStruct((batch, out_dim), jnp.float32),
    )(user_emb, tm3, ts_pad, Wu.T, mode_lut, W_ts)


# speedup vs baseline: 1.0061x; 1.0061x over previous
import jax
import jax.numpy as jnp
from jax import lax
from jax.experimental import pallas as pl
from jax.experimental.pallas import tpu as pltpu
from jax.experimental.pallas import tpu_sc as plsc

_NC = 2
_NS = 16
_NW = _NC * _NS
_CHUNK = 128
_LANES = 128


def _sc_gather_body(idx_hbm, table_hbm, out_hbm, idx_v, rows_v, sem):
    wid = lax.axis_index("s") * _NC + lax.axis_index("c")
    n_chunks = idx_v.shape[0]
    rows_per_w = n_chunks * _CHUNK
    pltpu.sync_copy(idx_hbm.at[wid], idx_v)
    copies = [
        pltpu.async_copy(
            table_hbm.at[idx_v.at[c]],
            rows_v.at[pl.ds(c * _CHUNK, _CHUNK)],
            sem,
        )
        for c in range(n_chunks)
    ]
    for cp in copies:
        cp.wait()
    pltpu.sync_copy(rows_v, out_hbm.at[pl.ds(wid * rows_per_w, rows_per_w)])


def _sc_gather_lines(line_idx, table2):
    batch = line_idx.shape[0]
    rows_per_w = batch // _NW
    n_chunks = rows_per_w // _CHUNK
    idx3 = line_idx.reshape(_NW, n_chunks, _CHUNK)
    mesh = plsc.VectorSubcoreMesh(core_axis_name="c", subcore_axis_name="s")
    gather = pl.kernel(
        _sc_gather_body,
        out_type=jax.ShapeDtypeStruct((batch, _LANES), jnp.float32),
        mesh=mesh,
        scratch_types=[
            pltpu.VMEM((n_chunks, _CHUNK), jnp.int32),
            pltpu.VMEM((rows_per_w, _LANES), jnp.float32),
            pltpu.SemaphoreType.DMA,
        ],
    )
    return gather(idx3, table2)


def _tc_body(ul_ref, sub_ref, tm_ref, ts_ref, wu_ref, mc_ref, wt_ref, o_ref):
    dim = wu_ref.shape[0]
    lines = ul_ref[...]
    sub = sub_ref[0, 0, :][:, None]
    u = lines[:, 3 * dim:4 * dim]
    for k in (2, 1, 0):
        u = jnp.where(sub == k, lines[:, k * dim:(k + 1) * dim], u)
    tm = tm_ref[0, 0, :]
    n_modes = mc_ref.shape[0]
    onehot = (
        tm[:, None] == lax.broadcasted_iota(jnp.int32, (1, n_modes), 1)
    ).astype(jnp.float32)
    acc = jnp.dot(u, wu_ref[...], preferred_element_type=jnp.float32)
    acc += jnp.dot(onehot, mc_ref[...], preferred_element_type=jnp.float32)
    acc += jnp.dot(ts_ref[...], wt_ref[...], preferred_element_type=jnp.float32)
    o_ref[...] = acc


def kernel(user_id, transport_mode, timestamp, user_table, mode_table,
           W_time, b_time, W_pref, b_pref):
    batch = user_id.shape[0]
    dim = user_table.shape[1]
    out_dim = W_pref.shape[0]
    n_modes = mode_table.shape[0]
    t_in = timestamp.shape[1]
    per_line = _LANES // dim

    Wu = W_pref[:, :dim]
    Wm = W_pref[:, dim:2 * dim]
    Wt = W_pref[:, 2 * dim:3 * dim]
    mode_lut = mode_table @ Wm.T + b_pref + b_time @ Wt.T
    t_pad = 8
    W_ts = jnp.zeros((t_pad, out_dim), jnp.float32).at[:t_in].set((Wt @ W_time).T)
    ts_pad = jnp.zeros((batch, t_pad), jnp.float32).at[:, :t_in].set(timestamp)

    uid = user_id.astype(jnp.int32)
    table2 = user_table.reshape(user_table.shape[0] // per_line, _LANES)
    user_lines = _sc_gather_lines(uid // per_line, table2)

    blk = 2048
    n_blk = batch // blk
    sub3 = (uid % per_line).reshape(n_blk, 1, blk)
    tm3 = transport_mode.astype(jnp.int32).reshape(n_blk, 1, blk)

    return pl.pallas_call(
        _tc_body,
        grid=(n_blk,),
        in_specs=[
            pl.BlockSpec((blk, _LANES), lambda i: (i, 0)),
            pl.BlockSpec((1, 1, blk), lambda i: (i, 0, 0)),
            pl.BlockSpec((1, 1, blk), lambda i: (i, 0, 0)),
            pl.BlockSpec((blk, t_pad), lambda i: (i, 0)),
            pl.BlockSpec((dim, out_dim), lambda i: (0, 0)),
            pl.BlockSpec((n_modes, out_dim), lambda i: (0, 0)),
            pl.BlockSpec((t_pad, out_dim), lambda i: (0, 0)),
        ],
        out_specs=pl.BlockSpec((blk, out_dim), lambda i: (i, 0)),
        out_shape=jax.ShapeDtypeStruct((batch, out_dim), jnp.float32),
    )(user_lines, sub3, tm3, ts_pad, Wu.T, mode_lut, W_ts)
